# baseline (speedup 1.0000x reference)
import jax
import jax.numpy as jnp
from jax import lax
from jax.experimental import pallas as pl
from jax.experimental.pallas import tpu as pltpu
from jax.experimental.pallas import tpu_sc as plsc

N = 10000
E = 320000
D = 128
DA = 144
NC = 2
NS = 16
CH = 128
EPT = 10240
NCHUNK = EPT // CH
NCH0 = 40
NCH1 = 2 * NCHUNK - NCH0
NPAD = 10240
ROWS_PT = NPAD // NS
BLK = 400
GRID = N // BLK


def _make_sc_agg(width, tc_tiling=False):
  mesh = plsc.VectorSubcoreMesh(core_axis_name="c", subcore_axis_name="s")
  out_type = [jax.ShapeDtypeStruct((NC * NPAD, width), jnp.float32)]
  scratch = [
      pltpu.VMEM((CH,), jnp.int32),
      pltpu.VMEM((CH,), jnp.int32),
      pltpu.VMEM((CH,), jnp.int32),
      pltpu.VMEM((CH,), jnp.int32),
      pltpu.VMEM((CH, width), jnp.float32),
      pltpu.VMEM((CH, width), jnp.float32),
      pltpu.VMEM_SHARED((NPAD, width), jnp.float32),
      pltpu.SemaphoreType.DMA,
      pltpu.SemaphoreType.DMA,
      pltpu.SemaphoreType.DMA,
      pltpu.SemaphoreType.DMA,
  ]

  def body(table_hbm, src_hbm, dst_hbm, agg_out, src_v0, dst_v0, src_v1,
           dst_v1, rows_v0, rows_v1, agg_sh, g0, g1, s0, s1):
    cid = lax.axis_index("c")
    sid = lax.axis_index("s")
    nch = jnp.where(cid == 0, NCH0, NCH1)
    e0 = jnp.where(cid == 0, sid * NCH0, NS * NCH0 + sid * NCH1) * CH

    def idx_load(c, sv, dv):
      pltpu.sync_copy(src_hbm.at[pl.ds(e0 + c * CH, CH)], sv)
      pltpu.sync_copy(dst_hbm.at[pl.ds(e0 + c * CH, CH)], dv)

    def gstart(sv, rv, sem):
      pltpu.async_copy(table_hbm.at[sv], rv, sem)

    def gwait(sv, rv, sem):
      pltpu.make_async_copy(table_hbm.at[sv], rv, sem).wait()

    def sstart(rv, dv, sem):
      pltpu.async_copy(rv, agg_sh.at[dv], sem, add=True)

    def swait(rv, dv, sem):
      pltpu.make_async_copy(rv, agg_sh.at[dv], sem).wait()

    def zrow(r, carry):
      for j in range(width // 16):
        rows_v0[r, pl.ds(j * 16, 16)] = jnp.zeros((16,), jnp.float32)
      return carry
    lax.fori_loop(0, CH, zrow, 0)

    base = sid * ROWS_PT
    for k in range(ROWS_PT // CH):
      pltpu.sync_copy(rows_v0, agg_sh.at[pl.ds(base + k * CH, CH)])
    plsc.subcore_barrier()

    idx_load(0, src_v0, dst_v0)
    gstart(src_v0, rows_v0, g0)

    def pair(p, carry):
      c0 = 2 * p

      @pl.when(p > 0)
      def _():
        swait(rows_v1, dst_v1, s1)
      idx_load(c0 + 1, src_v1, dst_v1)
      gstart(src_v1, rows_v1, g1)

      gwait(src_v0, rows_v0, g0)
      sstart(rows_v0, dst_v0, s0)

      @pl.when(c0 + 2 < nch)
      def _():
        swait(rows_v0, dst_v0, s0)
        idx_load(c0 + 2, src_v0, dst_v0)
        gstart(src_v0, rows_v0, g0)

      gwait(src_v1, rows_v1, g1)
      sstart(rows_v1, dst_v1, s1)
      return carry
    lax.fori_loop(0, nch // 2, pair, 0)
    swait(rows_v0, dst_v0, s0)
    swait(rows_v1, dst_v1, s1)
    plsc.subcore_barrier()

    pltpu.sync_copy(agg_sh.at[pl.ds(base, ROWS_PT)],
                    agg_out.at[pl.ds(cid * NPAD + base, ROWS_PT)])

  return pl.kernel(body, out_type=out_type, mesh=mesh, scratch_types=scratch,
                   compiler_params=pltpu.CompilerParams(
                       use_tc_tiling_on_sc=tc_tiling))


_sc_agg_aug = _make_sc_agg(DA)
_sc_agg = _make_sc_agg(D, tc_tiling=True)


_DN = (((1,), (1,)), ((), ()))


def _tc1_body(x_ref, wl_ref, wr_ref, b_ref, a_ref, r_ref):
  xb = x_ref[...]
  al = lax.dot_general(xb, wl_ref[...], _DN, preferred_element_type=jnp.float32)
  a_ref[...] = jnp.concatenate(
      [al, jnp.ones((BLK, 1), jnp.float32),
       jnp.zeros((BLK, DA - D - 1), jnp.float32)], axis=1)
  r_ref[...] = lax.dot_general(xb, wr_ref[...], _DN,
                               preferred_element_type=jnp.float32) + b_ref[...]


def _tc2_body(aggp_ref, r1_ref, wl_ref, wr_ref, b_ref, a2_ref, r2_ref):
  agg = aggp_ref[0, :, 0:D] + aggp_ref[1, :, 0:D]
  cnt = aggp_ref[0, :, D:D + 1] + aggp_ref[1, :, D:D + 1]
  h = jnp.maximum(agg / jnp.maximum(cnt, 1.0) + r1_ref[...], 0.0)
  a2_ref[...] = lax.dot_general(h, wl_ref[...], _DN,
                                preferred_element_type=jnp.float32)
  r2_ref[...] = lax.dot_general(h, wr_ref[...], _DN,
                                preferred_element_type=jnp.float32) + b_ref[...]


def _tc3_body(aggp2_ref, aggp1_ref, r2_ref, o_ref):
  agg = aggp2_ref[0] + aggp2_ref[1]
  cnt = aggp1_ref[0, :, D:D + 1] + aggp1_ref[1, :, D:D + 1]
  o_ref[...] = agg / jnp.maximum(cnt, 1.0) + r2_ref[...]


_row_spec = pl.BlockSpec((BLK, D), lambda i: (i, 0))
_arow_spec = pl.BlockSpec((BLK, DA), lambda i: (i, 0))
_w_spec = pl.BlockSpec((D, D), lambda i: (0, 0))
_b_spec = pl.BlockSpec((1, D), lambda i: (0, 0))
_aggp_spec = pl.BlockSpec((NC, BLK, D), lambda i: (0, i, 0))
_aggpa_spec = pl.BlockSpec((NC, BLK, DA), lambda i: (0, i, 0))
_row_out = jax.ShapeDtypeStruct((N, D), jnp.float32)
_arow_out = jax.ShapeDtypeStruct((N, DA), jnp.float32)

_tc1 = pl.pallas_call(
    _tc1_body, grid=(GRID,),
    in_specs=[_row_spec, _w_spec, _w_spec, _b_spec],
    out_specs=[_arow_spec, _row_spec],
    out_shape=[_arow_out, _row_out])

_tc2 = pl.pallas_call(
    _tc2_body, grid=(GRID,),
    in_specs=[_aggpa_spec, _row_spec, _w_spec, _w_spec, _b_spec],
    out_specs=[_row_spec, _row_spec],
    out_shape=[_row_out, _row_out])

_tc3 = pl.pallas_call(
    _tc3_body, grid=(GRID,),
    in_specs=[_aggp_spec, _aggpa_spec, _row_spec],
    out_specs=_row_spec,
    out_shape=_row_out)


@jax.jit
def kernel(x, edge_index, W1l, W1r, b1, W2l, W2r, b2):
  epad = NC * NS * EPT
  src = jnp.concatenate([edge_index[0], jnp.zeros((epad - E,), jnp.int32)])
  dst = jnp.concatenate([edge_index[1], jnp.full((epad - E,), N, jnp.int32)])
  b1r = b1.reshape(1, D)
  b2r = b2.reshape(1, D)

  a1, r1 = _tc1(x, W1l, W1r, b1r)
  (aggp1,) = _sc_agg_aug(a1, src, dst)
  aggp1 = aggp1.reshape(NC, NPAD, DA)
  a2, r2 = _tc2(aggp1, r1, W2l, W2r, b2r)
  (aggp2,) = _sc_agg(a2, src, dst)
  aggp2 = aggp2.reshape(NC, NPAD, D)
  return _tc3(aggp2, aggp1, r2)


# speedup vs baseline: 1.1840x; 1.1840x over previous
import jax
import jax.numpy as jnp
from jax import lax
from jax.experimental import pallas as pl
from jax.experimental.pallas import tpu as pltpu
from jax.experimental.pallas import tpu_sc as plsc

N = 10000
E = 320000
D = 128
DA = 144
NC = 2
NS = 16
CH = 128
EPT = 10240
NCHUNK = EPT // CH
NCH0 = 120
NCH1 = 2 * NCHUNK - NCH0
NPAD = 10240
ROWS_PT = NPAD // NS
BLK = 400
GRID = N // BLK


def _make_sc_agg(width, tc_tiling=False):
  mesh = plsc.VectorSubcoreMesh(core_axis_name="c", subcore_axis_name="s")
  out_type = [jax.ShapeDtypeStruct((NC * NPAD, width), jnp.float32)]
  scratch = [
      pltpu.VMEM((CH,), jnp.int32),
      pltpu.VMEM((CH,), jnp.int32),
      pltpu.VMEM((CH,), jnp.int32),
      pltpu.VMEM((CH,), jnp.int32),
      pltpu.VMEM((CH, width), jnp.float32),
      pltpu.VMEM((CH, width), jnp.float32),
      pltpu.VMEM_SHARED((NPAD, width), jnp.float32),
      pltpu.SemaphoreType.DMA,
      pltpu.SemaphoreType.DMA,
      pltpu.SemaphoreType.DMA,
      pltpu.SemaphoreType.DMA,
  ]

  def body(table_hbm, src_hbm, dst_hbm, agg_out, src_v0, dst_v0, src_v1,
           dst_v1, rows_v0, rows_v1, agg_sh, g0, g1, s0, s1):
    cid = lax.axis_index("c")
    sid = lax.axis_index("s")
    nch = jnp.where(cid == 0, NCH0, NCH1)
    e0 = jnp.where(cid == 0, sid * NCH0, NS * NCH0 + sid * NCH1) * CH

    def idx_load(c, sv, dv):
      pltpu.sync_copy(src_hbm.at[pl.ds(e0 + c * CH, CH)], sv)
      pltpu.sync_copy(dst_hbm.at[pl.ds(e0 + c * CH, CH)], dv)

    def gstart(sv, rv, sem):
      pltpu.async_copy(table_hbm.at[sv], rv, sem)

    def gwait(sv, rv, sem):
      pltpu.make_async_copy(table_hbm.at[sv], rv, sem).wait()

    def sstart(rv, dv, sem):
      pltpu.async_copy(rv, agg_sh.at[dv], sem, add=True)

    def swait(rv, dv, sem):
      pltpu.make_async_copy(rv, agg_sh.at[dv], sem).wait()

    def zrow(r, carry):
      for j in range(width // 16):
        rows_v0[r, pl.ds(j * 16, 16)] = jnp.zeros((16,), jnp.float32)
      return carry
    lax.fori_loop(0, CH, zrow, 0)

    base = sid * ROWS_PT
    for k in range(ROWS_PT // CH):
      pltpu.sync_copy(rows_v0, agg_sh.at[pl.ds(base + k * CH, CH)])
    plsc.subcore_barrier()

    idx_load(0, src_v0, dst_v0)
    gstart(src_v0, rows_v0, g0)

    def pair(p, carry):
      c0 = 2 * p

      @pl.when(p > 0)
      def _():
        swait(rows_v1, dst_v1, s1)
      idx_load(c0 + 1, src_v1, dst_v1)
      gstart(src_v1, rows_v1, g1)

      gwait(src_v0, rows_v0, g0)
      sstart(rows_v0, dst_v0, s0)

      @pl.when(c0 + 2 < nch)
      def _():
        swait(rows_v0, dst_v0, s0)
        idx_load(c0 + 2, src_v0, dst_v0)
        gstart(src_v0, rows_v0, g0)

      gwait(src_v1, rows_v1, g1)
      sstart(rows_v1, dst_v1, s1)
      return carry
    lax.fori_loop(0, nch // 2, pair, 0)
    swait(rows_v0, dst_v0, s0)
    swait(rows_v1, dst_v1, s1)
    plsc.subcore_barrier()

    pltpu.sync_copy(agg_sh.at[pl.ds(base, ROWS_PT)],
                    agg_out.at[pl.ds(cid * NPAD + base, ROWS_PT)])

  return pl.kernel(body, out_type=out_type, mesh=mesh, scratch_types=scratch,
                   compiler_params=pltpu.CompilerParams(
                       use_tc_tiling_on_sc=tc_tiling))


_sc_agg_aug = _make_sc_agg(DA)
_sc_agg = _make_sc_agg(D, tc_tiling=True)


_DN = (((1,), (1,)), ((), ()))


def _tc1_body(x_ref, wl_ref, wr_ref, b_ref, a_ref, r_ref):
  xb = x_ref[...]
  al = lax.dot_general(xb, wl_ref[...], _DN, preferred_element_type=jnp.float32)
  a_ref[...] = jnp.concatenate(
      [al, jnp.ones((BLK, 1), jnp.float32),
       jnp.zeros((BLK, DA - D - 1), jnp.float32)], axis=1)
  r_ref[...] = lax.dot_general(xb, wr_ref[...], _DN,
                               preferred_element_type=jnp.float32) + b_ref[...]


def _tc2_body(aggp_ref, r1_ref, wl_ref, wr_ref, b_ref, a2_ref, r2_ref):
  agg = aggp_ref[0, :, 0:D] + aggp_ref[1, :, 0:D]
  cnt = aggp_ref[0, :, D:D + 1] + aggp_ref[1, :, D:D + 1]
  h = jnp.maximum(agg / jnp.maximum(cnt, 1.0) + r1_ref[...], 0.0)
  a2_ref[...] = lax.dot_general(h, wl_ref[...], _DN,
                                preferred_element_type=jnp.float32)
  r2_ref[...] = lax.dot_general(h, wr_ref[...], _DN,
                                preferred_element_type=jnp.float32) + b_ref[...]


def _tc3_body(aggp2_ref, aggp1_ref, r2_ref, o_ref):
  agg = aggp2_ref[0] + aggp2_ref[1]
  cnt = aggp1_ref[0, :, D:D + 1] + aggp1_ref[1, :, D:D + 1]
  o_ref[...] = agg / jnp.maximum(cnt, 1.0) + r2_ref[...]


_row_spec = pl.BlockSpec((BLK, D), lambda i: (i, 0))
_arow_spec = pl.BlockSpec((BLK, DA), lambda i: (i, 0))
_w_spec = pl.BlockSpec((D, D), lambda i: (0, 0))
_b_spec = pl.BlockSpec((1, D), lambda i: (0, 0))
_aggp_spec = pl.BlockSpec((NC, BLK, D), lambda i: (0, i, 0))
_aggpa_spec = pl.BlockSpec((NC, BLK, DA), lambda i: (0, i, 0))
_row_out = jax.ShapeDtypeStruct((N, D), jnp.float32)
_arow_out = jax.ShapeDtypeStruct((N, DA), jnp.float32)

_tc1 = pl.pallas_call(
    _tc1_body, grid=(GRID,),
    in_specs=[_row_spec, _w_spec, _w_spec, _b_spec],
    out_specs=[_arow_spec, _row_spec],
    out_shape=[_arow_out, _row_out])

_tc2 = pl.pallas_call(
    _tc2_body, grid=(GRID,),
    in_specs=[_aggpa_spec, _row_spec, _w_spec, _w_spec, _b_spec],
    out_specs=[_row_spec, _row_spec],
    out_shape=[_row_out, _row_out])

_tc3 = pl.pallas_call(
    _tc3_body, grid=(GRID,),
    in_specs=[_aggp_spec, _aggpa_spec, _row_spec],
    out_specs=_row_spec,
    out_shape=_row_out)


@jax.jit
def kernel(x, edge_index, W1l, W1r, b1, W2l, W2r, b2):
  epad = NC * NS * EPT
  src = jnp.concatenate([edge_index[0], jnp.zeros((epad - E,), jnp.int32)])
  dst = jnp.concatenate([edge_index[1], jnp.full((epad - E,), N, jnp.int32)])
  b1r = b1.reshape(1, D)
  b2r = b2.reshape(1, D)

  a1, r1 = _tc1(x, W1l, W1r, b1r)
  (aggp1,) = _sc_agg_aug(a1, src, dst)
  aggp1 = aggp1.reshape(NC, NPAD, DA)
  a2, r2 = _tc2(aggp1, r1, W2l, W2r, b2r)
  (aggp2,) = _sc_agg(a2, src, dst)
  aggp2 = aggp2.reshape(NC, NPAD, D)
  return _tc3(aggp2, aggp1, r2)
